# speedup vs baseline: 40.0785x; 40.0785x over previous
import functools

import jax
import jax.numpy as jnp
from jax import lax
from jax.experimental import pallas as pl
from jax.experimental.pallas import tpu as pltpu
from jax.experimental.pallas import tpu_sc as plsc

_POS_MIN = 0.7
_B, _C, _H, _W = 8, 1, 384, 384
_TOTAL = _B * _C * _H * _W
_NC, _NS, _L = 2, 16, 16
_NW = _NC * _NS
_PER_W = _TOTAL // _NW
_CH = 18432
_NCHUNK = _PER_W // _CH
_VECS = _CH // _L


def _sc_partials(gr, pr, ga, pa, cm):
    mesh = plsc.VectorSubcoreMesh(core_axis_name="c", subcore_axis_name="s")

    @functools.partial(
        pl.kernel,
        out_type=jax.ShapeDtypeStruct((_NW, 6, _L), jnp.float32),
        mesh=mesh,
        scratch_types=[pltpu.VMEM((_CH,), jnp.float32) for _ in range(5)]
        + [pltpu.VMEM((6, _L), jnp.float32)],
    )
    def k(gr_h, pr_h, ga_h, pa_h, cm_h, out_h, bgr, bpr, bga, bpa, bcm, obuf):
        wid = lax.axis_index("c") * _NS + lax.axis_index("s")
        base = wid * _PER_W

        def chunk_body(c, accs):
            off = base + c * _CH
            pltpu.sync_copy(gr_h.at[pl.ds(off, _CH)], bgr)
            pltpu.sync_copy(pr_h.at[pl.ds(off, _CH)], bpr)
            pltpu.sync_copy(ga_h.at[pl.ds(off, _CH)], bga)
            pltpu.sync_copy(pa_h.at[pl.ds(off, _CH)], bpa)
            pltpu.sync_copy(cm_h.at[pl.ds(off, _CH)], bcm)

            def body(i, a):
                tsr, psr, pcr, tsa, psa, pca = a
                s = pl.ds(i * _L, _L)
                g = bgr[s]
                p = bpr[s]
                h = bga[s]
                q = bpa[s]
                w = bcm[s]
                zero = jnp.zeros((_L,), jnp.float32)
                one = jnp.full((_L,), 1.0, jnp.float32)
                dr = g - p
                lr = dr * dr * w
                mr = g > _POS_MIN
                da = h - q
                la = da * da * w
                ma = h > _POS_MIN
                return (
                    tsr + lr,
                    psr + jnp.where(mr, lr, zero),
                    pcr + jnp.where(mr, one, zero),
                    tsa + la,
                    psa + jnp.where(ma, la, zero),
                    pca + jnp.where(ma, one, zero),
                )

            return lax.fori_loop(0, _VECS, body, accs)

        z = jnp.zeros((_L,), jnp.float32)
        accs = lax.fori_loop(0, _NCHUNK, chunk_body, (z, z, z, z, z, z))
        for j in range(6):
            obuf[j, :] = accs[j]
        pltpu.sync_copy(obuf, out_h.at[wid])

    return k(gr, pr, ga, pa, cm)


_ROWS = 9


def _hard_topk_sum(gt3, pred3, conf3, kf):
    def kern(kf_ref, g_ref, p_ref, c_ref, out_ref, neg_ref):
        for j in range(_ROWS):
            g = g_ref[j]
            d = g - p_ref[j]
            l = d * d * c_ref[j]
            neg_ref[j] = jnp.where(g > _POS_MIN, 0.0, l)
        kf_ = kf_ref[0]

        def cnt_ge(t):
            def b(j, acc):
                return acc + jnp.sum((neg_ref[j] >= t).astype(jnp.float32))

            return lax.fori_loop(0, _ROWS, b, jnp.float32(0.0))

        def bs(_, lohi):
            lo, hi = lohi
            mid = (lo + hi) // 2
            t = lax.bitcast_convert_type(mid, jnp.float32)
            ok = cnt_ge(t) >= kf_
            return (jnp.where(ok, mid, lo), jnp.where(ok, hi, mid))

        lo, _ = lax.fori_loop(
            0, 31, bs, (jnp.int32(0), jnp.int32(0x3F800001))
        )
        t = lax.bitcast_convert_type(lo, jnp.float32)

        def b2(j, acc):
            s, c = acc
            v = neg_ref[j]
            m = v > t
            return (
                s + jnp.sum(jnp.where(m, v, 0.0)),
                c + jnp.sum(m.astype(jnp.float32)),
            )

        s, c = lax.fori_loop(0, _ROWS, b2, (jnp.float32(0.0), jnp.float32(0.0)))
        out_ref[0] = jnp.where(kf_ > 0.0, s + (kf_ - c) * t, 0.0)

    res = pl.pallas_call(
        kern,
        out_shape=jax.ShapeDtypeStruct((1,), jnp.float32),
        in_specs=[
            pl.BlockSpec(memory_space=pltpu.SMEM),
            pl.BlockSpec(memory_space=pltpu.VMEM),
            pl.BlockSpec(memory_space=pltpu.VMEM),
            pl.BlockSpec(memory_space=pltpu.VMEM),
        ],
        out_specs=pl.BlockSpec(memory_space=pltpu.SMEM),
        scratch_shapes=[pltpu.VMEM((_ROWS, 128, 1024), jnp.float32)],
    )(jnp.reshape(kf, (1,)), gt3, pred3, conf3)
    return res[0]


def _one_loss(ts, ps, pc, gt3, pred3, conf3):
    total_f = jnp.float32(_TOTAL)

    def easy(_):
        return ts / total_f

    def hard(_):
        kf = 3.0 * pc
        topk = _hard_topk_sum(gt3, pred3, conf3, kf)
        return (topk + ps) / (4.0 * pc)

    return lax.cond(4.0 * pc >= total_f, easy, hard, operand=None)


def kernel(gt_region, pred_region, gt_affinity, pred_affinity, conf_map):
    flat = lambda a: jnp.reshape(a, (_TOTAL,))
    gr, pr, ga, pa, cm = (
        flat(gt_region),
        flat(pred_region),
        flat(gt_affinity),
        flat(pred_affinity),
        flat(conf_map),
    )
    partials = _sc_partials(gr, pr, ga, pa, cm)
    s = jnp.sum(partials, axis=(0, 2))

    r3 = lambda a: jnp.reshape(a, (_ROWS, 128, 1024))
    res_r = _one_loss(s[0], s[1], s[2], r3(gr), r3(pr), r3(cm))
    res_a = _one_loss(s[3], s[4], s[5], r3(ga), r3(pa), r3(cm))
    return res_r + res_a
